# Initial kernel scaffold; baseline (speedup 1.0000x reference)
import jax
import jax.numpy as jnp
from jax.experimental import pallas as pl


def kernel(x, boxes, box_labels, W_ff, b_ff, ln_g, ln_b, W_disr, b_disr, W_heads, b_heads):
    raise NotImplementedError("write your pallas kernel here")


# speedup vs baseline: 24.0140x; 24.0140x over previous
import functools

import jax
import jax.numpy as jnp
from jax import lax
from jax.experimental import pallas as pl
from jax.experimental.pallas import tpu as pltpu

_ID2CAT = tuple(int(2 + (i * 97) % 398) for i in range(120))
_MAX_ATT = 397
_NUM_ATTR = 120
_D = 768
_K = 512
_SIDE = 32
_SCALE = 32.0 / 512.0


def _pool_body(meta_ref, x_ref, out_ref):
    i = pl.program_id(0)
    b = meta_ref[i, 0]
    hs = meta_ref[i, 1]
    nr = meta_ref[i, 2]
    ws = meta_ref[i, 3]
    nc = meta_ref[i, 4]
    valid = (nr > 0) & (nc > 0)

    def body(j, acc):
        slab = x_ref[b, pl.ds((hs + j) * _SIDE, _SIDE), :]
        return jnp.maximum(acc, slab)

    acc = lax.fori_loop(0, nr, body,
                        jnp.full((_SIDE, _D), -jnp.inf, jnp.float32))
    xpos = lax.broadcasted_iota(jnp.int32, (_SIDE, _D), 0)
    acc = jnp.where((xpos >= ws) & (xpos < ws + nc), acc, -jnp.inf)
    red = jnp.max(acc, axis=0)
    out_ref[0, 0, :] = jnp.where(valid, red, 0.0)


def _roi_pool(meta, x):
    grid_spec = pltpu.PrefetchScalarGridSpec(
        num_scalar_prefetch=1,
        grid=(_K,),
        in_specs=[pl.BlockSpec(x.shape, lambda i, m: (0, 0, 0))],
        out_specs=pl.BlockSpec((1, 1, _D), lambda i, m: (i, 0, 0)),
    )
    out = pl.pallas_call(
        _pool_body,
        grid_spec=grid_spec,
        out_shape=jax.ShapeDtypeStruct((_K, 1, _D), jnp.float32),
    )(meta, x)
    return out.reshape(_K, _D)


def _ff_body(p_ref, wff_ref, bff_ref, g_ref, be_ref, wd_ref, bd_ref,
             h_ref, disr_ref):
    h0 = jnp.dot(p_ref[:], wff_ref[:], preferred_element_type=jnp.float32)
    h0 = h0 + bff_ref[:]
    mu = jnp.mean(h0, axis=-1, keepdims=True)
    var = jnp.mean((h0 - mu) ** 2, axis=-1, keepdims=True)
    hn = (h0 - mu) / jnp.sqrt(var + 1e-5) * g_ref[:] + be_ref[:]
    h = hn * 0.5 * (1.0 + lax.erf(hn / jnp.sqrt(jnp.float32(2.0))))
    h_ref[:] = h
    disr_ref[:] = jnp.dot(h, wd_ref[:], preferred_element_type=jnp.float32) + bd_ref[:]


def _ff(pooled, W_ff, b_ff, ln_g, ln_b, W_disr, b_disr):
    return pl.pallas_call(
        _ff_body,
        out_shape=(jax.ShapeDtypeStruct((_K, _D), jnp.float32),
                   jax.ShapeDtypeStruct((_K, 1), jnp.float32)),
    )(pooled, W_ff, b_ff.reshape(1, _D), ln_g.reshape(1, _D),
      ln_b.reshape(1, _D), W_disr, b_disr.reshape(1, 1))


def _heads_body(perm_ref, offs_ref, cats_ref, h_ref, w_ref, bh_ref,
                out_ref, rows_ref):
    e = pl.program_id(0)
    start = offs_ref[e]
    n = offs_ref[e + 1] - start
    cat = cats_ref[e]
    colmask = lax.broadcasted_iota(jnp.int32, (8, _MAX_ATT), 1) < cat

    def chunk(c, carry):
        base = start + c * 8
        rem = n - c * 8
        for j in range(8):
            src = perm_ref[base + jnp.minimum(j, rem - 1)]
            rows_ref[j, :] = h_ref[src, :]
        prod = jnp.dot(rows_ref[:], w_ref[0],
                       preferred_element_type=jnp.float32)
        prod = prod + bh_ref[0, 0]
        prod = jnp.where(colmask, prod, 0.0)
        for j in range(8):
            @pl.when(j < rem)
            def _():
                out_ref[perm_ref[base + j], :] = prod[j]
        return carry

    lax.fori_loop(0, (n + 7) // 8, chunk, 0)


def _heads(perm, offs, cats, h, W_heads, b_heads):
    grid_spec = pltpu.PrefetchScalarGridSpec(
        num_scalar_prefetch=3,
        grid=(_NUM_ATTR,),
        in_specs=[
            pl.BlockSpec((_K, _D), lambda e, p, o, c: (0, 0)),
            pl.BlockSpec((1, _D, _MAX_ATT), lambda e, p, o, c: (e, 0, 0)),
            pl.BlockSpec((1, 1, _MAX_ATT), lambda e, p, o, c: (e, 0, 0)),
        ],
        out_specs=pl.BlockSpec((_K, _MAX_ATT), lambda e, p, o, c: (0, 0)),
        scratch_shapes=[pltpu.VMEM((8, _D), jnp.float32)],
    )
    return pl.pallas_call(
        _heads_body,
        grid_spec=grid_spec,
        out_shape=jax.ShapeDtypeStruct((_K, _MAX_ATT), jnp.float32),
    )(perm, offs, cats, h, W_heads, b_heads.reshape(_NUM_ATTR, 1, _MAX_ATT))


def kernel(x, boxes, box_labels, W_ff, b_ff, ln_g, ln_b, W_disr, b_disr,
           W_heads, b_heads):
    q = jnp.round(boxes[:, 1:5].astype(jnp.float32) * _SCALE).astype(jnp.int32)
    x1, y1, x2, y2 = q[:, 0], q[:, 1], q[:, 2], q[:, 3]
    roi_w = jnp.maximum(x2 - x1 + 1, 1)
    roi_h = jnp.maximum(y2 - y1 + 1, 1)
    hs = jnp.clip(y1, 0, _SIDE)
    he = jnp.clip(y1 + roi_h, 0, _SIDE)
    ws = jnp.clip(x1, 0, _SIDE)
    we = jnp.clip(x1 + roi_w, 0, _SIDE)
    b = boxes[:, 0].astype(jnp.int32)
    meta = jnp.stack([b, hs, he - hs, ws, we - ws], axis=1)

    pooled = _roi_pool(meta, x)
    h, disr_logits = _ff(pooled, W_ff, b_ff, ln_g, ln_b, W_disr, b_disr)

    labels = box_labels.astype(jnp.int32)
    perm = jnp.argsort(labels).astype(jnp.int32)
    counts = jnp.zeros((_NUM_ATTR,), jnp.int32).at[labels].add(1)
    offs = jnp.concatenate([jnp.zeros((1,), jnp.int32),
                            jnp.cumsum(counts).astype(jnp.int32)])
    cats = jnp.asarray(_ID2CAT, dtype=jnp.int32)

    logits = _heads(perm, offs, cats, h, W_heads, b_heads)
    return (h, logits, disr_logits)


# speedup vs baseline: 33.2240x; 1.3835x over previous
import functools

import jax
import jax.numpy as jnp
from jax import lax
from jax.experimental import pallas as pl
from jax.experimental.pallas import tpu as pltpu
from jax.experimental.pallas import tpu_sc as plsc

_ID2CAT = tuple(int(2 + (i * 97) % 398) for i in range(120))
_MAX_ATT = 397
_NUM_ATTR = 120
_D = 768
_K = 512
_SIDE = 32
_SCALE = 32.0 / 512.0


_D16 = _D // 16


def _sc_pool_call(x2d, meta_flat):
    info = plsc.get_sparse_core_info()
    nc, ns = info.num_cores, info.num_subcores
    nw = nc * ns
    bpw = _K // nw
    mesh = plsc.VectorSubcoreMesh(core_axis_name="c", subcore_axis_name="s")

    @functools.partial(
        pl.kernel,
        mesh=mesh,
        out_type=jax.ShapeDtypeStruct((_K, _D), jnp.float32),
        scratch_types=[
            pltpu.VMEM((bpw * 16,), jnp.int32),
            pltpu.VMEM((16, _D), jnp.float32),
            pltpu.VMEM((_D,), jnp.float32),
            pltpu.SemaphoreType.DMA,
        ],
    )
    def k(x_hbm, meta_hbm, out_hbm, meta_v, rows_v, acc_v, sem):
        wid = lax.axis_index("s") * nc + lax.axis_index("c")
        base_box = wid * bpw
        pltpu.sync_copy(meta_hbm.at[pl.ds(base_box * 16, bpw * 16)], meta_v)
        lanes = lax.iota(jnp.int32, 16)

        def one_box(i, carry):
            mrow = meta_v[pl.ds(i * 16, 16)]
            base = mrow[0]
            n = mrow[1]
            ncol = mrow[2]
            magic = mrow[3]

            initval = jnp.where(n > 0, -jnp.inf, 0.0).astype(jnp.float32)

            def init_d(d, c):
                acc_v[pl.ds(d * 16, 16)] = jnp.full((16,), initval,
                                                    jnp.float32)
                return c

            lax.fori_loop(0, _D16, init_d, 0)

            def one_chunk(c, carry2):
                t = jnp.minimum(c * 16 + lanes, n - 1)
                yy = lax.shift_right_logical(t * magic, 16)
                ids = base + yy * _SIDE + (t - yy * ncol)
                pltpu.async_copy(x_hbm.at[ids], rows_v, sem).wait()

                def max_d(d, c3):
                    a = acc_v[pl.ds(d * 16, 16)]
                    for r in range(16):
                        a = jnp.maximum(a, rows_v[r, pl.ds(d * 16, 16)])
                    acc_v[pl.ds(d * 16, 16)] = a
                    return c3

                lax.fori_loop(0, _D16, max_d, 0)
                return carry2

            lax.fori_loop(0, (n + 15) // 16, one_chunk, 0)

            pltpu.sync_copy(acc_v, out_hbm.at[base_box + i])
            return carry

        lax.fori_loop(0, bpw, one_box, 0)

    return k(x2d, meta_flat)


def _pool_body(meta_ref, x_ref, out_ref):
    i = pl.program_id(0)
    b = meta_ref[i, 0]
    hs = meta_ref[i, 1]
    nr = meta_ref[i, 2]
    ws = meta_ref[i, 3]
    nc = meta_ref[i, 4]
    valid = (nr > 0) & (nc > 0)

    def body(j, acc):
        slab = x_ref[b, pl.ds((hs + j) * _SIDE, _SIDE), :]
        return jnp.maximum(acc, slab)

    acc = lax.fori_loop(0, nr, body,
                        jnp.full((_SIDE, _D), -jnp.inf, jnp.float32))
    xpos = lax.broadcasted_iota(jnp.int32, (_SIDE, _D), 0)
    acc = jnp.where((xpos >= ws) & (xpos < ws + nc), acc, -jnp.inf)
    red = jnp.max(acc, axis=0)
    out_ref[0, 0, :] = jnp.where(valid, red, 0.0)


def _roi_pool(meta, x):
    grid_spec = pltpu.PrefetchScalarGridSpec(
        num_scalar_prefetch=1,
        grid=(_K,),
        in_specs=[pl.BlockSpec(x.shape, lambda i, m: (0, 0, 0))],
        out_specs=pl.BlockSpec((1, 1, _D), lambda i, m: (i, 0, 0)),
    )
    out = pl.pallas_call(
        _pool_body,
        grid_spec=grid_spec,
        out_shape=jax.ShapeDtypeStruct((_K, 1, _D), jnp.float32),
    )(meta, x)
    return out.reshape(_K, _D)


def _ff_body(p_ref, wff_ref, bff_ref, g_ref, be_ref, wd_ref, bd_ref,
             h_ref, disr_ref):
    h0 = jnp.dot(p_ref[:], wff_ref[:], preferred_element_type=jnp.float32)
    h0 = h0 + bff_ref[:]
    mu = jnp.mean(h0, axis=-1, keepdims=True)
    var = jnp.mean((h0 - mu) ** 2, axis=-1, keepdims=True)
    hn = (h0 - mu) / jnp.sqrt(var + 1e-5) * g_ref[:] + be_ref[:]
    h = hn * 0.5 * (1.0 + lax.erf(hn / jnp.sqrt(jnp.float32(2.0))))
    h_ref[:] = h
    disr_ref[:] = jnp.dot(h, wd_ref[:], preferred_element_type=jnp.float32) + bd_ref[:]


def _ff(pooled, W_ff, b_ff, ln_g, ln_b, W_disr, b_disr):
    return pl.pallas_call(
        _ff_body,
        out_shape=(jax.ShapeDtypeStruct((_K, _D), jnp.float32),
                   jax.ShapeDtypeStruct((_K, 1), jnp.float32)),
    )(pooled, W_ff, b_ff.reshape(1, _D), ln_g.reshape(1, _D),
      ln_b.reshape(1, _D), W_disr, b_disr.reshape(1, 1))


def _heads_body(perm_ref, offs_ref, cats_ref, h_ref, w_ref, bh_ref,
                out_ref, rows_ref):
    e = pl.program_id(0)
    start = offs_ref[e]
    n = offs_ref[e + 1] - start
    cat = cats_ref[e]
    colmask = lax.broadcasted_iota(jnp.int32, (8, _MAX_ATT), 1) < cat

    def chunk(c, carry):
        base = start + c * 8
        rem = n - c * 8
        for j in range(8):
            src = perm_ref[base + jnp.minimum(j, rem - 1)]
            rows_ref[j, :] = h_ref[src, :]
        prod = jnp.dot(rows_ref[:], w_ref[0],
                       preferred_element_type=jnp.float32)
        prod = prod + bh_ref[0, 0]
        prod = jnp.where(colmask, prod, 0.0)
        for j in range(8):
            @pl.when(j < rem)
            def _():
                out_ref[perm_ref[base + j], :] = prod[j]
        return carry

    lax.fori_loop(0, (n + 7) // 8, chunk, 0)


def _heads(perm, offs, cats, h, W_heads, b_heads):
    grid_spec = pltpu.PrefetchScalarGridSpec(
        num_scalar_prefetch=3,
        grid=(_NUM_ATTR,),
        in_specs=[
            pl.BlockSpec((_K, _D), lambda e, p, o, c: (0, 0)),
            pl.BlockSpec((1, _D, _MAX_ATT), lambda e, p, o, c: (e, 0, 0)),
            pl.BlockSpec((1, 1, _MAX_ATT), lambda e, p, o, c: (e, 0, 0)),
        ],
        out_specs=pl.BlockSpec((_K, _MAX_ATT), lambda e, p, o, c: (0, 0)),
        scratch_shapes=[pltpu.VMEM((8, _D), jnp.float32)],
    )
    return pl.pallas_call(
        _heads_body,
        grid_spec=grid_spec,
        out_shape=jax.ShapeDtypeStruct((_K, _MAX_ATT), jnp.float32),
    )(perm, offs, cats, h, W_heads, b_heads.reshape(_NUM_ATTR, 1, _MAX_ATT))


def kernel(x, boxes, box_labels, W_ff, b_ff, ln_g, ln_b, W_disr, b_disr,
           W_heads, b_heads):
    q = jnp.round(boxes[:, 1:5].astype(jnp.float32) * _SCALE).astype(jnp.int32)
    x1, y1, x2, y2 = q[:, 0], q[:, 1], q[:, 2], q[:, 3]
    roi_w = jnp.maximum(x2 - x1 + 1, 1)
    roi_h = jnp.maximum(y2 - y1 + 1, 1)
    hs = jnp.clip(y1, 0, _SIDE)
    he = jnp.clip(y1 + roi_h, 0, _SIDE)
    ws = jnp.clip(x1, 0, _SIDE)
    we = jnp.clip(x1 + roi_w, 0, _SIDE)
    b = boxes[:, 0].astype(jnp.int32)
    nrows = he - hs
    ncols = we - ws
    base = b * (_SIDE * _SIDE) + hs * _SIDE + ws
    n = nrows * ncols
    magic = (65536 + jnp.maximum(ncols, 1) - 1) // jnp.maximum(ncols, 1)
    meta = jnp.stack(
        [base, n, ncols, magic] + [jnp.zeros_like(n)] * 12,
        axis=1)

    pooled = _sc_pool_call(x.reshape(-1, _D), meta.reshape(-1))
    h, disr_logits = _ff(pooled, W_ff, b_ff, ln_g, ln_b, W_disr, b_disr)

    labels = box_labels.astype(jnp.int32)
    perm = jnp.argsort(labels).astype(jnp.int32)
    counts = jnp.zeros((_NUM_ATTR,), jnp.int32).at[labels].add(1)
    offs = jnp.concatenate([jnp.zeros((1,), jnp.int32),
                            jnp.cumsum(counts).astype(jnp.int32)])
    cats = jnp.asarray(_ID2CAT, dtype=jnp.int32)

    logits = _heads(perm, offs, cats, h, W_heads, b_heads)
    return (h, logits, disr_logits)
